# baseline (speedup 1.0000x reference)
import functools

import jax
import jax.numpy as jnp
from jax import lax
from jax.experimental import pallas as pl
from jax.experimental.pallas import tpu as pltpu
from jax.experimental.pallas import tpu_sc as plsc

_BATCH = 4096
_SEQ = 200
_DIM = 64
_NC, _NS = 2, 16
_NW = _NC * _NS
_RPW = _BATCH // _NW
_SPLITS = (0, 104, 200)


def _gather_kernel(idx_hbm, table_hbm, out_hbm,
                   idx_v, buf0, buf1, gsem0, gsem1, ssem0, ssem1):
    wid = lax.axis_index("s") * _NC + lax.axis_index("c")
    base = wid * _RPW
    pltpu.sync_copy(idx_hbm.at[pl.ds(base, _RPW)], idx_v)

    def fire_gathers(r, buf, sem):
        for lo, hi in zip(_SPLITS[:-1], _SPLITS[1:]):
            pltpu.async_copy(
                table_hbm.at[idx_v.at[r, pl.ds(lo, hi - lo)]],
                buf.at[pl.ds(lo, hi - lo)], sem)

    def wait_gathers(buf, sem):
        pltpu.make_async_copy(table_hbm.at[pl.ds(0, _SEQ)], buf, sem).wait()

    def fire_store(r, buf, sem):
        pltpu.async_copy(buf, out_hbm.at[base + r], sem)

    def wait_store(buf, sem):
        pltpu.make_async_copy(buf, out_hbm.at[base], sem).wait()

    fire_gathers(0, buf0, gsem0)
    fire_gathers(1, buf1, gsem1)

    def body(p, carry):
        r = 2 * p
        wait_gathers(buf0, gsem0)
        fire_store(r, buf0, ssem0)
        wait_gathers(buf1, gsem1)
        fire_store(r + 1, buf1, ssem1)
        wait_store(buf0, ssem0)
        fire_gathers(r + 2, buf0, gsem0)
        wait_store(buf1, ssem1)
        fire_gathers(r + 3, buf1, gsem1)
        return carry

    lax.fori_loop(0, _RPW // 2 - 1, body, 0)

    wait_gathers(buf0, gsem0)
    fire_store(_RPW - 2, buf0, ssem0)
    wait_gathers(buf1, gsem1)
    fire_store(_RPW - 1, buf1, ssem1)
    wait_store(buf0, ssem0)
    wait_store(buf1, ssem1)


def kernel(token_ids, embedding_table):
    mesh = plsc.VectorSubcoreMesh(core_axis_name="c", subcore_axis_name="s")
    run = functools.partial(
        pl.kernel,
        mesh=mesh,
        out_type=jax.ShapeDtypeStruct((_BATCH, _SEQ, _DIM), jnp.float32),
        scratch_types=[
            pltpu.VMEM((_RPW, _SEQ), jnp.int32),
            pltpu.VMEM((_SEQ, _DIM), jnp.float32),
            pltpu.VMEM((_SEQ, _DIM), jnp.float32),
            pltpu.SemaphoreType.DMA,
            pltpu.SemaphoreType.DMA,
            pltpu.SemaphoreType.DMA,
            pltpu.SemaphoreType.DMA,
        ],
        compiler_params=pltpu.CompilerParams(use_tc_tiling_on_sc=False),
    )(_gather_kernel)
    return run(token_ids, embedding_table)


# speedup vs baseline: 1.0143x; 1.0143x over previous
import functools

import jax
import jax.numpy as jnp
from jax import lax
from jax.experimental import pallas as pl
from jax.experimental.pallas import tpu as pltpu
from jax.experimental.pallas import tpu_sc as plsc

_BATCH = 4096
_SEQ = 200
_DIM = 64
_NC, _NS = 2, 16
_NW = _NC * _NS
_LANES = 128
_ROWS = _BATCH * _SEQ // _LANES
_RPW = _ROWS // _NW
_K = 5
_CHUNKS = _RPW // _K
_CTOK = _K * _LANES
_TPW = _RPW * _LANES


def _gather_kernel(idx_hbm, table_hbm, out_hbm,
                   idx_v, buf0, buf1, gsem0, gsem1, ssem0, ssem1):
    wid = lax.axis_index("s") * _NC + lax.axis_index("c")
    base = wid * _RPW
    tok = wid * _TPW
    pltpu.sync_copy(idx_hbm.at[pl.ds(base, _RPW)], idx_v)

    def fire_gathers(c, buf, sem):
        for k in range(_K):
            pltpu.async_copy(
                table_hbm.at[idx_v.at[c * _K + k]],
                buf.at[pl.ds(k * _LANES, _LANES)], sem)

    def wait_gathers(buf, sem):
        pltpu.make_async_copy(out_hbm.at[pl.ds(tok, _CTOK)], buf, sem).wait()

    def fire_store(c, buf, sem):
        pltpu.async_copy(buf, out_hbm.at[pl.ds(tok + c * _CTOK, _CTOK)], sem)

    def wait_store(buf, sem):
        pltpu.make_async_copy(buf, out_hbm.at[pl.ds(tok, _CTOK)], sem).wait()

    fire_gathers(0, buf0, gsem0)
    fire_gathers(1, buf1, gsem1)

    def body(p, carry):
        c = 2 * p
        wait_gathers(buf0, gsem0)
        fire_store(c, buf0, ssem0)
        wait_gathers(buf1, gsem1)
        fire_store(c + 1, buf1, ssem1)
        wait_store(buf0, ssem0)
        fire_gathers(c + 2, buf0, gsem0)
        wait_store(buf1, ssem1)
        fire_gathers(c + 3, buf1, gsem1)
        return carry

    lax.fori_loop(0, _CHUNKS // 2 - 1, body, 0)

    wait_gathers(buf0, gsem0)
    fire_store(_CHUNKS - 2, buf0, ssem0)
    wait_gathers(buf1, gsem1)
    fire_store(_CHUNKS - 1, buf1, ssem1)
    wait_store(buf0, ssem0)
    wait_store(buf1, ssem1)


def kernel(token_ids, embedding_table):
    flat_ids = token_ids.reshape(_ROWS, _LANES)
    mesh = plsc.VectorSubcoreMesh(core_axis_name="c", subcore_axis_name="s")
    run = functools.partial(
        pl.kernel,
        mesh=mesh,
        out_type=jax.ShapeDtypeStruct((_BATCH * _SEQ, _DIM), jnp.float32),
        scratch_types=[
            pltpu.VMEM((_RPW, _LANES), jnp.int32),
            pltpu.VMEM((_CTOK, _DIM), jnp.float32),
            pltpu.VMEM((_CTOK, _DIM), jnp.float32),
            pltpu.SemaphoreType.DMA,
            pltpu.SemaphoreType.DMA,
            pltpu.SemaphoreType.DMA,
            pltpu.SemaphoreType.DMA,
        ],
        compiler_params=pltpu.CompilerParams(use_tc_tiling_on_sc=False),
    )(_gather_kernel)
    out = run(flat_ids, embedding_table)
    return out.reshape(_BATCH, _SEQ, _DIM)
